# baseline (speedup 1.0000x reference)
import jax
import jax.numpy as jnp
from jax import lax
from jax.experimental import pallas as pl
from jax.experimental.pallas import tpu as pltpu
from jax.experimental.pallas import tpu_sc as plsc

B = 16384
S = 200
D = 32
W0 = 96
W1 = S - W0
NC = 2
NS = 16
NW = NC * NS
BPW = B // NW
CB = 8
NCHUNK = BPW // CB


def _fire(g, wid, idx_hbm, table_hbm, idx_va, idx_vb, rows_v, sem):
    rbase = wid * BPW + g * CB
    pltpu.sync_copy(idx_hbm.at[pl.ds(rbase, CB), pl.ds(0, W0)], idx_va)
    pltpu.sync_copy(idx_hbm.at[pl.ds(rbase, CB), pl.ds(W0, W1)], idx_vb)
    for b in range(CB):
        pltpu.async_copy(table_hbm.at[idx_va.at[b]],
                         rows_v.at[pl.ds(b * S, W0)], sem)
        pltpu.async_copy(table_hbm.at[idx_vb.at[b]],
                         rows_v.at[pl.ds(b * S + W0, W1)], sem)


def _drain(idx_va, idx_vb, table_hbm, rows_v, sem):
    for b in range(CB):
        pltpu.make_async_copy(table_hbm.at[idx_va.at[b]],
                              rows_v.at[pl.ds(b * S, W0)], sem).wait()
        pltpu.make_async_copy(table_hbm.at[idx_vb.at[b]],
                              rows_v.at[pl.ds(b * S + W0, W1)], sem).wait()


def _compute(g, wid, idx_va, idx_vb, rows_v, out_v, out_hbm):
    lanes = lax.iota(jnp.int32, 16)
    zf = jnp.zeros((16,), jnp.float32)
    zi = jnp.zeros((16,), jnp.int32)
    ones = jnp.ones((16,), jnp.int32)
    eight = jnp.full((16,), 8, jnp.int32)
    for b in range(CB):
        base = b * S

        def body(s, accs):
            a0, a1 = accs
            a0 = a0 + rows_v[base + s, 0:16]
            a1 = a1 + rows_v[base + s, 16:32]
            return a0, a1

        a0, a1 = lax.fori_loop(0, S, body, (zf, zf), unroll=8)

        cv = zi
        for k in range(6):
            chunk = idx_va[b, pl.ds(k * 16, 16)]
            cv = cv + jnp.where(chunk != zi, ones, zi)
        for k in range(6):
            chunk = idx_vb[b, pl.ds(k * 16, 16)]
            cv = cv + jnp.where(chunk != zi, ones, zi)
        rem = idx_vb[b, 88:104]
        cv = cv + jnp.where((lanes >= eight) & (rem != zi), ones, zi)
        cntv = jnp.full((16,), jnp.sum(cv).astype(jnp.float32), jnp.float32)
        rv = jnp.ones((16,), jnp.float32) / cntv
        out_v[b, 0:16] = a0 * rv
        out_v[b, 16:32] = a1 * rv
    pltpu.sync_copy(out_v, out_hbm.at[pl.ds(wid * BPW + g * CB, CB)])


def _sc_kernel(idx_hbm, table_hbm, out_hbm,
               idx_aa, idx_ab, idx_ba, idx_bb, rows_a, rows_b, out_v,
               sem_a, sem_b):
    wid = lax.axis_index("s") * NC + lax.axis_index("c")
    _fire(0, wid, idx_hbm, table_hbm, idx_aa, idx_ab, rows_a, sem_a)

    def outer(i, carry):
        g0 = 2 * i
        g1 = g0 + 1
        _fire(g1, wid, idx_hbm, table_hbm, idx_ba, idx_bb, rows_b, sem_b)
        _drain(idx_aa, idx_ab, table_hbm, rows_a, sem_a)
        _compute(g0, wid, idx_aa, idx_ab, rows_a, out_v, out_hbm)

        @pl.when(g1 + 1 < NCHUNK)
        def _():
            _fire(g1 + 1, wid, idx_hbm, table_hbm, idx_aa, idx_ab, rows_a, sem_a)

        _drain(idx_ba, idx_bb, table_hbm, rows_b, sem_b)
        _compute(g1, wid, idx_ba, idx_bb, rows_b, out_v, out_hbm)
        return carry

    lax.fori_loop(0, NCHUNK // 2, outer, 0)


@jax.jit
def kernel(input_sequences, table):
    mesh = plsc.VectorSubcoreMesh(core_axis_name="c", subcore_axis_name="s",
                                  num_cores=NC, num_subcores=NS)
    f = pl.kernel(
        _sc_kernel,
        out_type=jax.ShapeDtypeStruct((B, D), jnp.float32),
        mesh=mesh,
        compiler_params=pltpu.CompilerParams(needs_layout_passes=False,
                                             use_tc_tiling_on_sc=False),
        scratch_types=[
            pltpu.VMEM((CB, W0), jnp.int32),
            pltpu.VMEM((CB, W1), jnp.int32),
            pltpu.VMEM((CB, W0), jnp.int32),
            pltpu.VMEM((CB, W1), jnp.int32),
            pltpu.VMEM((CB * S, D), jnp.float32),
            pltpu.VMEM((CB * S, D), jnp.float32),
            pltpu.VMEM((CB, D), jnp.float32),
            pltpu.SemaphoreType.DMA,
            pltpu.SemaphoreType.DMA,
        ],
    )
    return f(input_sequences.astype(jnp.int32), table)


# speedup vs baseline: 1.0684x; 1.0684x over previous
import jax
import jax.numpy as jnp
from jax import lax
from jax.experimental import pallas as pl
from jax.experimental.pallas import tpu as pltpu
from jax.experimental.pallas import tpu_sc as plsc

B = 16384
S = 200
D = 32
NC = 2
NS = 16
NW = NC * NS
BPW = B // NW
SCB = 16
NSUP = BPW // SCB
CB = 8
IR = SCB * S // 128
PKC = 256


def _mesh():
    return plsc.VectorSubcoreMesh(core_axis_name="c", subcore_axis_name="s",
                                  num_cores=NC, num_subcores=NS)


def _wid():
    return lax.axis_index("s") * NC + lax.axis_index("c")


def _pack_kernel(idx_hbm, out_hbm, vin, vout, sem):
    wid = _wid()
    for c in range(BPW // PKC):

        def chunk(r0):
            pltpu.sync_copy(idx_hbm.at[pl.ds(r0, PKC)], vin)

            def row(r, carry):
                for k in range(12):
                    vout[pl.ds(S * r + 16 * k, 16)] = vin[r, pl.ds(16 * k, 16)]
                vout[pl.ds(S * r + 184, 16)] = vin[r, pl.ds(184, 16)]
                return carry

            lax.fori_loop(0, PKC, row, 0, unroll=2)
            pltpu.sync_copy(vout, out_hbm.at[pl.ds(r0 * S, PKC * S)])

        chunk(wid * BPW + c * PKC)


def _fire(sub, idx2_v, table_hbm, rows_v, sem):
    if sub == 0:
        for j in range(12):
            pltpu.async_copy(table_hbm.at[idx2_v.at[j]],
                             rows_v.at[pl.ds(128 * j, 128)], sem)
        pltpu.async_copy(table_hbm.at[idx2_v.at[12, pl.ds(0, 64)]],
                         rows_v.at[pl.ds(1536, 64)], sem)
    else:
        pltpu.async_copy(table_hbm.at[idx2_v.at[12, pl.ds(64, 64)]],
                         rows_v.at[pl.ds(0, 64)], sem)
        for j in range(12):
            pltpu.async_copy(table_hbm.at[idx2_v.at[13 + j]],
                             rows_v.at[pl.ds(64 + 128 * j, 128)], sem)


def _drain(sub, idx2_v, table_hbm, rows_v, sem):
    if sub == 0:
        for j in range(12):
            pltpu.make_async_copy(table_hbm.at[idx2_v.at[j]],
                                  rows_v.at[pl.ds(128 * j, 128)], sem).wait()
        pltpu.make_async_copy(table_hbm.at[idx2_v.at[12, pl.ds(0, 64)]],
                              rows_v.at[pl.ds(1536, 64)], sem).wait()
    else:
        pltpu.make_async_copy(table_hbm.at[idx2_v.at[12, pl.ds(64, 64)]],
                              rows_v.at[pl.ds(0, 64)], sem).wait()
        for j in range(12):
            pltpu.make_async_copy(table_hbm.at[idx2_v.at[13 + j]],
                                  rows_v.at[pl.ds(64 + 128 * j, 128)], sem).wait()


def _count_windows(o):
    lo, hi = S * o, S * (o + 1)
    wins = []
    for r in range(lo // 128, (hi + 127) // 128):
        c_lo, c_hi = max(0, lo - 128 * r), min(128, hi - 128 * r)
        pos = c_lo
        while pos < c_hi:
            if pos + 16 <= c_hi:
                wins.append((r, pos, 0))
                pos += 16
            else:
                wins.append((r, c_hi - 16, 16 - (c_hi - pos)))
                pos = c_hi
    return wins


def _compute(sub, idx2_v, rows_v, out_v):
    lanes = lax.iota(jnp.int32, 16)
    zf = jnp.zeros((16,), jnp.float32)
    zi = jnp.zeros((16,), jnp.int32)
    ones = jnp.ones((16,), jnp.int32)
    for b in range(CB):
        base = b * S

        def body(s, accs):
            a0, a1 = accs
            a0 = a0 + rows_v[base + s, 0:16]
            a1 = a1 + rows_v[base + s, 16:32]
            return a0, a1

        a0, a1 = lax.fori_loop(0, S, body, (zf, zf), unroll=8)

        o = sub * CB + b
        cv = zi
        for r, c, nmask in _count_windows(o):
            chunk = idx2_v[r, pl.ds(c, 16)]
            nz = chunk != zi
            if nmask:
                nz = nz & (lanes >= jnp.full((16,), nmask, jnp.int32))
            cv = cv + jnp.where(nz, ones, zi)
        cntv = jnp.full((16,), jnp.sum(cv).astype(jnp.float32), jnp.float32)
        rv = jnp.ones((16,), jnp.float32) / cntv
        out_v[o, 0:16] = a0 * rv
        out_v[o, 16:32] = a1 * rv


def _sc_kernel(idx2_hbm, table_hbm, out_hbm,
               idx2_a, idx2_b, rows_a, rows_b, out_v, sem_a, sem_b):
    wid = _wid()

    def load_idx(i, idx2_v):
        pltpu.sync_copy(idx2_hbm.at[pl.ds((wid * NSUP + i) * IR, IR)], idx2_v)

    load_idx(0, idx2_a)
    _fire(0, idx2_a, table_hbm, rows_a, sem_a)

    def halfstep(i, idx2_c, idx2_n, last):
        _fire(1, idx2_c, table_hbm, rows_b, sem_b)

        @pl.when(jnp.logical_not(last))
        def _():
            load_idx(i + 1, idx2_n)

        _drain(0, idx2_c, table_hbm, rows_a, sem_a)
        _compute(0, idx2_c, rows_a, out_v)

        @pl.when(jnp.logical_not(last))
        def _():
            _fire(0, idx2_n, table_hbm, rows_a, sem_a)

        _drain(1, idx2_c, table_hbm, rows_b, sem_b)
        _compute(1, idx2_c, rows_b, out_v)
        pltpu.sync_copy(out_v, out_hbm.at[pl.ds(wid * BPW + SCB * i, SCB)])

    def outer(t, carry):
        i0 = 2 * t
        halfstep(i0, idx2_a, idx2_b, jnp.bool_(False))
        halfstep(i0 + 1, idx2_b, idx2_a, i0 + 2 >= NSUP)
        return carry

    lax.fori_loop(0, NSUP // 2, outer, 0)


@jax.jit
def kernel(input_sequences, table):
    idx = input_sequences.astype(jnp.int32)
    pack = pl.kernel(
        _pack_kernel,
        out_type=jax.ShapeDtypeStruct((B * S,), jnp.int32),
        mesh=_mesh(),
        compiler_params=pltpu.CompilerParams(needs_layout_passes=False,
                                             use_tc_tiling_on_sc=True),
        scratch_types=[
            pltpu.VMEM((PKC, S), jnp.int32),
            pltpu.VMEM((PKC * S,), jnp.int32),
            pltpu.SemaphoreType.DMA,
        ],
    )
    idx_flat = pack(idx)
    idx2 = idx_flat.reshape(B * S // 128, 128)

    f = pl.kernel(
        _sc_kernel,
        out_type=jax.ShapeDtypeStruct((B, D), jnp.float32),
        mesh=_mesh(),
        compiler_params=pltpu.CompilerParams(needs_layout_passes=False,
                                             use_tc_tiling_on_sc=False),
        scratch_types=[
            pltpu.VMEM((IR, 128), jnp.int32),
            pltpu.VMEM((IR, 128), jnp.int32),
            pltpu.VMEM((CB * S, D), jnp.float32),
            pltpu.VMEM((CB * S, D), jnp.float32),
            pltpu.VMEM((SCB, D), jnp.float32),
            pltpu.SemaphoreType.DMA,
            pltpu.SemaphoreType.DMA,
        ],
    )
    return f(idx2, table)
